# baseline (speedup 1.0000x reference)
import jax
import jax.numpy as jnp
from jax import lax
from jax.experimental import pallas as pl
from jax.experimental.pallas import tpu as pltpu
from jax.experimental.pallas import tpu_sc as plsc

_BS, _T, _PN, _PT = 1024, 200, 32, 1
_D = 8
_N = _BS * _T * _PN * _PT
_NC, _NS, _L = 2, 16, 16
_NOCT = _T // 8
_NB = 32
_NU = _NOCT * 2
_ROW = 2048


def _sc_body(spk_hbm, tab_hbm, out_hbm,
             tab_v, in0, in1, ot0, ot1, si0, si1, so0, so1):
    cid = lax.axis_index("c")
    sub = lax.axis_index("s")
    bq = cid * 4 + (sub >> 2)
    qo = (sub & 3) * _NB
    b0 = bq * 128 + qo

    pltpu.sync_copy(tab_hbm, tab_v)

    lane = lax.iota(jnp.int32, _L)
    brow = [(lane + _L * bh) * 1024 for bh in range(2)]
    inb, outb = (in0, in1), (ot0, ot1)
    sib, sob = (si0, si1), (so0, so1)

    def in_slice(u):
        oct_, lt = u >> 1, u & 1
        return spk_hbm.at[pl.ds(oct_ * 8, 8), pl.ds(lt * 16, 16),
                          pl.ds(pl.multiple_of(bq * 128, 128), 128)]

    def compute(idx_v, out_v):
        @plsc.parallel_loop(0, 8 * 16, 1, unroll=2)
        def _(r):
            t_l = r >> 4
            p_l = r & 15
            cbase = t_l * 128 + p_l * _D
            for bh in range(2):
                spk = idx_v[t_l, p_l,
                            pl.ds(pl.multiple_of(qo + bh * _L, _L), _L)]
                addr = spk * _D
                gathered = [plsc.load_gather(tab_v, [addr + d])
                            for d in range(_D)]
                for d in range(_D):
                    plsc.store_scatter(out_v, [brow[bh] + (cbase + d)],
                                       gathered[d])

    def store_strips(out_v, u, sem):
        oct_, lt = u >> 1, u & 1
        for b in range(_NB):
            dst = out_hbm.at[pl.ds(
                pl.multiple_of((b0 + b) * (_NOCT * _ROW) + oct_ * _ROW
                               + lt * 1024, 1024), 1024)]
            pltpu.async_copy(out_v.at[pl.ds(b * 1024, 1024)], dst, sem)

    def drain_strips(out_v, sem):
        for b in range(_NB):
            pltpu.make_async_copy(
                out_v.at[pl.ds(b * 1024, 1024)],
                out_hbm.at[pl.ds(0, 1024)], sem).wait()

    pltpu.async_copy(in_slice(0), inb[0], sib[0])

    def pair_body(u2, carry):
        for p in range(2):
            u = u2 * 2 + p
            pltpu.make_async_copy(in_slice(0), inb[p], sib[p]).wait()

            @pl.when(u2 * 2 + p + 1 < _NU)
            def _():
                pltpu.async_copy(in_slice(u + 1), inb[1 - p], sib[1 - p])

            @pl.when(u2 * 2 + p >= 2)
            def _():
                drain_strips(outb[p], sob[p])

            compute(inb[p], outb[p])
            store_strips(outb[p], u, sob[p])
        return carry

    lax.fori_loop(0, _NU // 2, pair_body, 0)
    drain_strips(outb[0], sob[0])
    drain_strips(outb[1], sob[1])


def kernel(spikes, table):
    spk_t = jnp.transpose(spikes, (1, 2, 3, 0)).reshape(_T, _PN, _BS)
    tab_flat = table.reshape(21 * _D)

    mesh = plsc.VectorSubcoreMesh(core_axis_name="c", subcore_axis_name="s")
    out_flat = pl.kernel(
        _sc_body,
        out_type=jax.ShapeDtypeStruct((_N * _D,), jnp.float32),
        mesh=mesh,
        scratch_types=[
            pltpu.VMEM((21 * _D,), jnp.float32),
            pltpu.VMEM((8, 16, 128), jnp.int32),
            pltpu.VMEM((8, 16, 128), jnp.int32),
            pltpu.VMEM((_NB * 1024,), jnp.float32),
            pltpu.VMEM((_NB * 1024,), jnp.float32),
            pltpu.SemaphoreType.DMA,
            pltpu.SemaphoreType.DMA,
            pltpu.SemaphoreType.DMA,
            pltpu.SemaphoreType.DMA,
        ],
        compiler_params=pltpu.CompilerParams(needs_layout_passes=False),
    )(spk_t, tab_flat)
    out5 = out_flat.reshape(_BS, _T // 8, 2, 8, 128)
    return out5.transpose(0, 1, 3, 2, 4).reshape(_BS, _T, _PN * _PT * _D)


# speedup vs baseline: 3.0690x; 3.0690x over previous
import jax
import jax.numpy as jnp
from jax import lax
from jax.experimental import pallas as pl
from jax.experimental.pallas import tpu as pltpu
from jax.experimental.pallas import tpu_sc as plsc

_BS, _T, _PN, _PT = 1024, 200, 32, 1
_D = 8
_N = _BS * _T * _PN * _PT
_NC, _NS, _L = 2, 16, 16
_NW = _NC * _NS
_N_W = _N // _NW
_CHUNK = 4096
_NCHUNK = _N_W // _CHUNK


def _sc_body(spk_hbm, tab_hbm, out_hbm,
             tab_v, idx0, idx1, out0, out1, si0, si1, so0, so1):
    wid = lax.axis_index("s") * _NC + lax.axis_index("c")
    base = wid * _N_W

    pltpu.sync_copy(tab_hbm, tab_v)

    lane = lax.iota(jnp.int32, _L)
    scatter_base = [lane * _D + d for d in range(_D)]
    nstep = _CHUNK // _L
    idxb, outb = (idx0, idx1), (out0, out1)
    sib, sob = (si0, si1), (so0, so1)

    def idx_slice(c):
        return spk_hbm.at[pl.ds(base + c * _CHUNK, _CHUNK)]

    def out_slice(c):
        return out_hbm.at[pl.ds((base + c * _CHUNK) * _D, _CHUNK * _D)]

    def compute(idx_v, out_v):
        def offset(i):
            return (i >> 4) * 2048 + (i & 1) * 1024 + ((i >> 1) & 7) * 128

        @plsc.parallel_loop(0, nstep, 1, unroll=8)
        def _(i):
            spk = idx_v[pl.ds(i * _L, _L)]
            addr = spk * _D
            o = offset(i)
            dst = out_v.at[pl.ds(pl.multiple_of(o, 128), _L * _D)]
            gathered = [plsc.load_gather(tab_v, [addr + d]) for d in range(_D)]
            for d in range(_D):
                plsc.store_scatter(dst, [scatter_base[d]], gathered[d])

    pltpu.async_copy(idx_slice(0), idxb[0].at[pl.ds(0, _CHUNK)], sib[0])

    def pair_body(c2, carry):
        for b in range(2):
            c = c2 * 2 + b
            pltpu.make_async_copy(idx_slice(0), idxb[b].at[pl.ds(0, _CHUNK)],
                                  sib[b]).wait()

            @pl.when(c + 1 < _NCHUNK)
            def _():
                pltpu.async_copy(idx_slice(c + 1),
                                 idxb[1 - b].at[pl.ds(0, _CHUNK)], sib[1 - b])

            @pl.when(c >= 2)
            def _():
                pltpu.make_async_copy(outb[b], out_slice(0), sob[b]).wait()

            compute(idxb[b], outb[b])
            pltpu.async_copy(outb[b], out_slice(c), sob[b])
        return carry

    lax.fori_loop(0, _NCHUNK // 2, pair_body, 0)

    pltpu.make_async_copy(outb[0], out_slice(0), sob[0]).wait()
    pltpu.make_async_copy(outb[1], out_slice(0), sob[1]).wait()


def kernel(spikes, table):
    spk_flat = spikes.reshape(_N)
    tab_flat = table.reshape(21 * _D)

    mesh = plsc.VectorSubcoreMesh(core_axis_name="c", subcore_axis_name="s")
    out_flat = pl.kernel(
        _sc_body,
        out_type=jax.ShapeDtypeStruct((_N * _D,), jnp.float32),
        mesh=mesh,
        scratch_types=[
            pltpu.VMEM((21 * _D,), jnp.float32),
            pltpu.VMEM((_CHUNK + _L,), jnp.int32),
            pltpu.VMEM((_CHUNK + _L,), jnp.int32),
            pltpu.VMEM((_CHUNK * _D,), jnp.float32),
            pltpu.VMEM((_CHUNK * _D,), jnp.float32),
            pltpu.SemaphoreType.DMA,
            pltpu.SemaphoreType.DMA,
            pltpu.SemaphoreType.DMA,
            pltpu.SemaphoreType.DMA,
        ],
        compiler_params=pltpu.CompilerParams(needs_layout_passes=False),
    )(spk_flat, tab_flat)
    out5 = out_flat.reshape(_BS, _T // 8, 2, 8, 128)
    return out5.transpose(0, 1, 3, 2, 4).reshape(_BS, _T, _PN * _PT * _D)


# speedup vs baseline: 3.1337x; 1.0211x over previous
import jax
import jax.numpy as jnp
from jax import lax
from jax.experimental import pallas as pl
from jax.experimental.pallas import tpu as pltpu
from jax.experimental.pallas import tpu_sc as plsc

_BS, _T, _PN, _PT = 1024, 200, 32, 1
_D = 8
_N = _BS * _T * _PN * _PT
_NC, _NS, _L = 2, 16, 16
_NW = _NC * _NS
_N_W = _N // _NW
_CHUNK = 4096
_NCHUNK = _N_W // _CHUNK


def _sc_body(spk_hbm, tab_hbm, out_hbm,
             tab_v, idx0, idx1, out0, out1, si0, si1, so0, so1):
    wid = lax.axis_index("s") * _NC + lax.axis_index("c")
    base = wid * _N_W

    pltpu.sync_copy(tab_hbm, tab_v)

    lane = lax.iota(jnp.int32, _L)
    scatter_base = [lane * _D + d for d in range(_D)]
    nstep = _CHUNK // _L
    idxb, outb = (idx0, idx1), (out0, out1)
    sib, sob = (si0, si1), (so0, so1)

    def idx_slice(c):
        return spk_hbm.at[pl.ds(base + c * _CHUNK, _CHUNK)]

    def out_slice(c):
        return out_hbm.at[pl.ds((base + c * _CHUNK) * _D, _CHUNK * _D)]

    def compute(idx_v, out_v):
        def offset(i):
            return (i >> 4) * 2048 + (i & 1) * 1024 + ((i >> 1) & 7) * 128

        @plsc.parallel_loop(0, nstep, 1, unroll=16)
        def _(i):
            spk = idx_v[pl.ds(i * _L, _L)]
            addr = spk * _D
            o = offset(i)
            dst = out_v.at[pl.ds(pl.multiple_of(o, 128), _L * _D)]
            gathered = [plsc.load_gather(tab_v, [addr + d]) for d in range(_D)]
            for d in range(_D):
                plsc.store_scatter(dst, [scatter_base[d]], gathered[d])

    pltpu.async_copy(idx_slice(0), idxb[0].at[pl.ds(0, _CHUNK)], sib[0])

    def pair_body(c2, carry):
        for b in range(2):
            c = c2 * 2 + b
            pltpu.make_async_copy(idx_slice(0), idxb[b].at[pl.ds(0, _CHUNK)],
                                  sib[b]).wait()

            @pl.when(c + 1 < _NCHUNK)
            def _():
                pltpu.async_copy(idx_slice(c + 1),
                                 idxb[1 - b].at[pl.ds(0, _CHUNK)], sib[1 - b])

            @pl.when(c >= 2)
            def _():
                pltpu.make_async_copy(outb[b], out_slice(0), sob[b]).wait()

            compute(idxb[b], outb[b])
            pltpu.async_copy(outb[b], out_slice(c), sob[b])
        return carry

    lax.fori_loop(0, _NCHUNK // 2, pair_body, 0)

    pltpu.make_async_copy(outb[0], out_slice(0), sob[0]).wait()
    pltpu.make_async_copy(outb[1], out_slice(0), sob[1]).wait()


def kernel(spikes, table):
    spk_flat = spikes.reshape(_N)
    tab_flat = table.reshape(21 * _D)

    mesh = plsc.VectorSubcoreMesh(core_axis_name="c", subcore_axis_name="s")
    out_flat = pl.kernel(
        _sc_body,
        out_type=jax.ShapeDtypeStruct((_N * _D,), jnp.float32),
        mesh=mesh,
        scratch_types=[
            pltpu.VMEM((21 * _D,), jnp.float32),
            pltpu.VMEM((_CHUNK + _L,), jnp.int32),
            pltpu.VMEM((_CHUNK + _L,), jnp.int32),
            pltpu.VMEM((_CHUNK * _D,), jnp.float32),
            pltpu.VMEM((_CHUNK * _D,), jnp.float32),
            pltpu.SemaphoreType.DMA,
            pltpu.SemaphoreType.DMA,
            pltpu.SemaphoreType.DMA,
            pltpu.SemaphoreType.DMA,
        ],
        compiler_params=pltpu.CompilerParams(needs_layout_passes=False),
    )(spk_flat, tab_flat)
    out5 = out_flat.reshape(_BS, _T // 8, 2, 8, 128)
    return out5.transpose(0, 1, 3, 2, 4).reshape(_BS, _T, _PN * _PT * _D)


# speedup vs baseline: 3.1647x; 1.0099x over previous
import jax
import jax.numpy as jnp
from jax import lax
from jax.experimental import pallas as pl
from jax.experimental.pallas import tpu as pltpu
from jax.experimental.pallas import tpu_sc as plsc

_BS, _T, _PN, _PT = 1024, 200, 32, 1
_D = 8
_N = _BS * _T * _PN * _PT
_NC, _NS, _L = 2, 16, 16
_NW = _NC * _NS
_N_W = _N // _NW
_CHUNK = 4096
_NCHUNK = _N_W // _CHUNK


def _sc_body(spk_hbm, tab_hbm, out_hbm,
             tab_v, idx0, idx1, out0, out1, si0, si1, so0, so1):
    wid = lax.axis_index("s") * _NC + lax.axis_index("c")
    base = wid * _N_W

    pltpu.sync_copy(tab_hbm, tab_v)

    lane = lax.iota(jnp.int32, _L)
    scatter_base = [lane * _D + d for d in range(_D)]
    nstep = _CHUNK // _L
    idxb, outb = (idx0, idx1), (out0, out1)
    sib, sob = (si0, si1), (so0, so1)

    def idx_slice(c):
        return spk_hbm.at[pl.ds(base + c * _CHUNK, _CHUNK)]

    def out_slice(c):
        return out_hbm.at[pl.ds((base + c * _CHUNK) * _D, _CHUNK * _D)]

    def compute(idx_v, out_v):
        def offset(i):
            return (i >> 4) * 2048 + (i & 1) * 1024 + ((i >> 1) & 7) * 128

        @plsc.parallel_loop(0, nstep, 1, unroll=32)
        def _(i):
            spk = idx_v[pl.ds(i * _L, _L)]
            addr = spk * _D
            o = offset(i)
            dst = out_v.at[pl.ds(pl.multiple_of(o, 128), _L * _D)]
            gathered = [plsc.load_gather(tab_v, [addr + d]) for d in range(_D)]
            for d in range(_D):
                plsc.store_scatter(dst, [scatter_base[d]], gathered[d])

    pltpu.async_copy(idx_slice(0), idxb[0].at[pl.ds(0, _CHUNK)], sib[0])

    def pair_body(c2, carry):
        for b in range(2):
            c = c2 * 2 + b
            pltpu.make_async_copy(idx_slice(0), idxb[b].at[pl.ds(0, _CHUNK)],
                                  sib[b]).wait()

            @pl.when(c + 1 < _NCHUNK)
            def _():
                pltpu.async_copy(idx_slice(c + 1),
                                 idxb[1 - b].at[pl.ds(0, _CHUNK)], sib[1 - b])

            @pl.when(c >= 2)
            def _():
                pltpu.make_async_copy(outb[b], out_slice(0), sob[b]).wait()

            compute(idxb[b], outb[b])
            pltpu.async_copy(outb[b], out_slice(c), sob[b])
        return carry

    lax.fori_loop(0, _NCHUNK // 2, pair_body, 0)

    pltpu.make_async_copy(outb[0], out_slice(0), sob[0]).wait()
    pltpu.make_async_copy(outb[1], out_slice(0), sob[1]).wait()


def kernel(spikes, table):
    spk_flat = spikes.reshape(_N)
    tab_flat = table.reshape(21 * _D)

    mesh = plsc.VectorSubcoreMesh(core_axis_name="c", subcore_axis_name="s")
    out_flat = pl.kernel(
        _sc_body,
        out_type=jax.ShapeDtypeStruct((_N * _D,), jnp.float32),
        mesh=mesh,
        scratch_types=[
            pltpu.VMEM((21 * _D,), jnp.float32),
            pltpu.VMEM((_CHUNK + _L,), jnp.int32),
            pltpu.VMEM((_CHUNK + _L,), jnp.int32),
            pltpu.VMEM((_CHUNK * _D,), jnp.float32),
            pltpu.VMEM((_CHUNK * _D,), jnp.float32),
            pltpu.SemaphoreType.DMA,
            pltpu.SemaphoreType.DMA,
            pltpu.SemaphoreType.DMA,
            pltpu.SemaphoreType.DMA,
        ],
        compiler_params=pltpu.CompilerParams(needs_layout_passes=False),
    )(spk_flat, tab_flat)
    out5 = out_flat.reshape(_BS, _T // 8, 2, 8, 128)
    return out5.transpose(0, 1, 3, 2, 4).reshape(_BS, _T, _PN * _PT * _D)
